# baseline (speedup 1.0000x reference)
import jax
import jax.numpy as jnp
from jax import lax
from jax.experimental import pallas as pl
from jax.experimental.pallas import tpu as pltpu
from jax.experimental.pallas import tpu_sc as plsc

NC = 2
NS = 16
NW = NC * NS
N_PAD = 10240
RPS = N_PAD // NS
RPW = N_PAD // NW
D = 128
CH = 64
NCH = 16
BLK = CH * NCH


def _accum_body(xp, cs2, vs2, rs2, out,
                acc, csb, vsb, rsb, g0, g1, c0, c1,
                sg0, sg1, ss0, ss1):
    c = lax.axis_index("c")
    s = lax.axis_index("s")
    w = s * NC + c
    lane = lax.iota(jnp.int32, 16)
    nblk = cs2.shape[0] // (NW * NCH)
    rowbase0 = w * (nblk * NCH)
    gbufs = (g0, g1)
    cbufs = (c0, c1)
    sgs = (sg0, sg1)
    sss = (ss0, ss1)
    zeros16 = jnp.zeros((16,), jnp.float32)

    def zb_body(i, carry):
        r = jnp.full((16,), i // 8, jnp.int32)
        col = jnp.full((16,), (i % 8) * 16, jnp.int32) + lane
        plsc.store_scatter(g0, [r, col], zeros16)
        return carry
    lax.fori_loop(0, (CH * D) // 16, zb_body, 0)
    for j in range(RPS // CH):
        pltpu.sync_copy(g0, acc.at[pl.ds(s * RPS + j * CH, CH)])
    plsc.subcore_barrier()

    def blk_body(b, carry):
        rowbase = rowbase0 + b * NCH
        pltpu.sync_copy(cs2.at[pl.ds(rowbase, NCH)], csb)
        pltpu.sync_copy(vs2.at[pl.ds(rowbase, NCH)], vsb)
        pltpu.sync_copy(rs2.at[pl.ds(rowbase, NCH)], rsb)

        gather_pending = {}
        scatter_pending = {}
        gather_pending[0] = pltpu.async_copy(
            xp.at[csb.at[0]], gbufs[0], sgs[0])

        for k in range(NCH):
            a = k % 2
            if k + 1 < NCH:
                gather_pending[1 - a] = pltpu.async_copy(
                    xp.at[csb.at[k + 1]], gbufs[1 - a], sgs[1 - a])
            gather_pending.pop(a).wait()
            if a in scatter_pending:
                scatter_pending.pop(a).wait()

            g = gbufs[a]
            cb = cbufs[a]
            kvec = jnp.full((16,), k, jnp.int32)

            def e_body(i, carry2):
                base = jnp.full((16,), i * 4, jnp.int32)
                for u in range(4):
                    iv = base + u
                    vbc = plsc.load_gather(vsb, [kvec, iv])
                    for f in range(D // 16):
                        cf = lane + (f * 16)
                        gv = plsc.load_gather(g, [iv, cf])
                        plsc.store_scatter(cb, [iv, cf], gv * vbc)
                return carry2
            lax.fori_loop(0, CH // 4, e_body, 0)

            scatter_pending[a] = pltpu.async_copy(
                cb, acc.at[rsb.at[k]], sss[a], add=True)
        for a in sorted(scatter_pending):
            scatter_pending.pop(a).wait()
        return carry
    lax.fori_loop(0, nblk, blk_body, 0)

    plsc.subcore_barrier()
    out_pending = {}
    for j in range(RPS // CH):
        a = j % 2
        if a in out_pending:
            out_pending.pop(a).wait()
        gb = gbufs[a]
        pltpu.sync_copy(acc.at[pl.ds(s * RPS + j * CH, CH)], gb)
        out_pending[a] = pltpu.async_copy(
            gb, out.at[c].at[pl.ds(s * RPS + j * CH, CH)], sgs[a])
    for a in sorted(out_pending):
        out_pending.pop(a).wait()


def _combine_body(pf, x0f, xnf, bufa, bufb, bufx, s0, s1, s2):
    c = lax.axis_index("c")
    s = lax.axis_index("s")
    w = s * NC + c
    off = pl.multiple_of(w * (RPW * D), 8)
    cpa = pltpu.async_copy(pf.at[0].at[pl.ds(off, RPW * D)], bufa, s0)
    cpb = pltpu.async_copy(pf.at[1].at[pl.ds(off, RPW * D)], bufb, s1)
    cpx = pltpu.async_copy(x0f.at[pl.ds(off, RPW * D)], bufx, s2)
    cpa.wait()
    cpb.wait()
    cpx.wait()

    def r_body(i, carry):
        for u in range(2):
            ds = pl.ds((i * 2 + u) * 16, 16)
            av = bufa[ds]
            bv = bufb[ds]
            xv = bufx[ds]
            bufa[ds] = jnp.where(xv != 0.0, xv, av + bv)
        return carry
    lax.fori_loop(0, (RPW * D) // 32, r_body, 0)
    pltpu.sync_copy(bufa, xnf.at[pl.ds(off, RPW * D)])


def kernel(x, adj_indices, adj_values, mask, iter):
    n, d = x.shape
    row = adj_indices[0].astype(jnp.int32)
    col = adj_indices[1].astype(jnp.int32)
    vals = adj_values.astype(jnp.float32)
    e = row.shape[0]

    ep = ((e + NW * BLK - 1) // (NW * BLK)) * (NW * BLK)
    cs2 = jnp.zeros((ep,), jnp.int32).at[:e].set(col).reshape(ep // CH, CH)
    rs2 = jnp.zeros((ep,), jnp.int32).at[:e].set(row).reshape(ep // CH, CH)
    vs2 = jnp.zeros((ep,), jnp.float32).at[:e].set(vals).reshape(ep // CH, CH)

    x0 = jnp.where(mask != 0, x.astype(jnp.float32), 0.0)
    x0f = jnp.zeros((N_PAD * d,), jnp.float32).at[: n * d].set(x0.reshape(-1))
    x_pad = jnp.zeros((N_PAD, d), jnp.float32).at[:n].set(x.astype(jnp.float32))

    accum = pl.kernel(
        _accum_body,
        out_type=jax.ShapeDtypeStruct((NC, N_PAD, D), jnp.float32),
        mesh=plsc.VectorSubcoreMesh(core_axis_name="c", subcore_axis_name="s"),
        compiler_params=pltpu.CompilerParams(needs_layout_passes=False),
        scratch_types=[
            pltpu.VMEM_SHARED((N_PAD, D), jnp.float32),
            pltpu.VMEM((NCH, CH), jnp.int32),
            pltpu.VMEM((NCH, CH), jnp.float32),
            pltpu.VMEM((NCH, CH), jnp.int32),
            pltpu.VMEM((CH, D), jnp.float32),
            pltpu.VMEM((CH, D), jnp.float32),
            pltpu.VMEM((CH, D), jnp.float32),
            pltpu.VMEM((CH, D), jnp.float32),
            pltpu.SemaphoreType.DMA,
            pltpu.SemaphoreType.DMA,
            pltpu.SemaphoreType.DMA,
            pltpu.SemaphoreType.DMA,
        ],
    )

    combine = pl.kernel(
        _combine_body,
        out_type=jax.ShapeDtypeStruct((N_PAD * D,), jnp.float32),
        mesh=plsc.VectorSubcoreMesh(core_axis_name="c", subcore_axis_name="s"),
        compiler_params=pltpu.CompilerParams(needs_layout_passes=False),
        scratch_types=[
            pltpu.VMEM((RPW * D,), jnp.float32),
            pltpu.VMEM((RPW * D,), jnp.float32),
            pltpu.VMEM((RPW * D,), jnp.float32),
            pltpu.SemaphoreType.DMA,
            pltpu.SemaphoreType.DMA,
            pltpu.SemaphoreType.DMA,
        ],
    )

    def body(i, xp):
        partials = accum(xp, cs2, vs2, rs2)
        xnf = combine(partials.reshape(NC, N_PAD * D), x0f)
        return xnf.reshape(N_PAD, D)

    xf = lax.fori_loop(0, iter, body, x_pad)
    return xf[:n].astype(x.dtype)


# speedup vs baseline: 1.0694x; 1.0694x over previous
import jax
import jax.numpy as jnp
from jax import lax
from jax.experimental import pallas as pl
from jax.experimental.pallas import tpu as pltpu
from jax.experimental.pallas import tpu_sc as plsc

NC = 2
NS = 16
NW = NC * NS
N_PAD = 10240
RPS = N_PAD // NS
RPW = N_PAD // NW
D = 128
CH = 64
NCH = 8
BLK = CH * NCH


def _accum_body(xp, cs2, vs2, rs2, out,
                acc, csb, vsb, rsb, g0, g1, c0, c1,
                sg0, sg1, ss0, ss1):
    c = lax.axis_index("c")
    s = lax.axis_index("s")
    w = s * NC + c
    lane = lax.iota(jnp.int32, 16)
    nblk = cs2.shape[0] // (NW * NCH)
    rowbase0 = w * (nblk * NCH)
    gbufs = (g0, g1)
    cbufs = (c0, c1)
    sgs = (sg0, sg1)
    sss = (ss0, ss1)
    zeros16 = jnp.zeros((16,), jnp.float32)

    @plsc.parallel_loop(0, (CH * D) // 16, unroll=4)
    def zb_body(i):
        r = jnp.full((16,), i // 8, jnp.int32)
        col = jnp.full((16,), (i % 8) * 16, jnp.int32) + lane
        plsc.store_scatter(g0, [r, col], zeros16)
    for j in range(RPS // CH):
        pltpu.sync_copy(g0, acc.at[pl.ds(s * RPS + j * CH, CH)])
    plsc.subcore_barrier()

    def blk_body(b, carry):
        rowbase = rowbase0 + b * NCH
        pltpu.sync_copy(cs2.at[pl.ds(rowbase, NCH)], csb)
        pltpu.sync_copy(vs2.at[pl.ds(rowbase, NCH)], vsb)
        pltpu.sync_copy(rs2.at[pl.ds(rowbase, NCH)], rsb)

        gather_pending = {}
        scatter_pending = {}
        gather_pending[0] = pltpu.async_copy(
            xp.at[csb.at[0]], gbufs[0], sgs[0])

        for k in range(NCH):
            a = k % 2
            if k + 1 < NCH:
                gather_pending[1 - a] = pltpu.async_copy(
                    xp.at[csb.at[k + 1]], gbufs[1 - a], sgs[1 - a])
            gather_pending.pop(a).wait()
            if a in scatter_pending:
                scatter_pending.pop(a).wait()

            g = gbufs[a]
            cb = cbufs[a]
            kvec = jnp.full((16,), k, jnp.int32)

            @plsc.parallel_loop(0, CH // 4, unroll=2)
            def e_body(i):
                base = jnp.full((16,), i * 4, jnp.int32)
                for u in range(4):
                    iv = base + u
                    vbc = plsc.load_gather(vsb, [kvec, iv])
                    for f in range(D // 16):
                        cf = lane + (f * 16)
                        gv = plsc.load_gather(g, [iv, cf])
                        plsc.store_scatter(cb, [iv, cf], gv * vbc)

            scatter_pending[a] = pltpu.async_copy(
                cb, acc.at[rsb.at[k]], sss[a], add=True)
        for a in sorted(scatter_pending):
            scatter_pending.pop(a).wait()
        return carry
    lax.fori_loop(0, nblk, blk_body, 0)

    plsc.subcore_barrier()
    out_pending = {}
    for j in range(RPS // CH):
        a = j % 2
        if a in out_pending:
            out_pending.pop(a).wait()
        gb = gbufs[a]
        pltpu.sync_copy(acc.at[pl.ds(s * RPS + j * CH, CH)], gb)
        out_pending[a] = pltpu.async_copy(
            gb, out.at[c].at[pl.ds(s * RPS + j * CH, CH)], sgs[a])
    for a in sorted(out_pending):
        out_pending.pop(a).wait()


def _combine_body(pf, x0f, xnf, bufa, bufb, bufx, s0, s1, s2):
    c = lax.axis_index("c")
    s = lax.axis_index("s")
    w = s * NC + c
    off = pl.multiple_of(w * (RPW * D), 8)
    cpa = pltpu.async_copy(pf.at[0].at[pl.ds(off, RPW * D)], bufa, s0)
    cpb = pltpu.async_copy(pf.at[1].at[pl.ds(off, RPW * D)], bufb, s1)
    cpx = pltpu.async_copy(x0f.at[pl.ds(off, RPW * D)], bufx, s2)
    cpa.wait()
    cpb.wait()
    cpx.wait()

    @plsc.parallel_loop(0, (RPW * D) // 32, unroll=2)
    def r_body(i):
        for u in range(2):
            ds = pl.ds((i * 2 + u) * 16, 16)
            av = bufa[ds]
            bv = bufb[ds]
            xv = bufx[ds]
            bufa[ds] = jnp.where(xv != 0.0, xv, av + bv)
    pltpu.sync_copy(bufa, xnf.at[pl.ds(off, RPW * D)])


def kernel(x, adj_indices, adj_values, mask, iter):
    n, d = x.shape
    row = adj_indices[0].astype(jnp.int32)
    col = adj_indices[1].astype(jnp.int32)
    vals = adj_values.astype(jnp.float32)
    e = row.shape[0]

    ep = ((e + NW * BLK - 1) // (NW * BLK)) * (NW * BLK)
    cs2 = jnp.zeros((ep,), jnp.int32).at[:e].set(col).reshape(ep // CH, CH)
    rs2 = jnp.zeros((ep,), jnp.int32).at[:e].set(row).reshape(ep // CH, CH)
    vs2 = jnp.zeros((ep,), jnp.float32).at[:e].set(vals).reshape(ep // CH, CH)

    x0 = jnp.where(mask != 0, x.astype(jnp.float32), 0.0)
    x0f = jnp.zeros((N_PAD * d,), jnp.float32).at[: n * d].set(x0.reshape(-1))
    x_pad = jnp.zeros((N_PAD, d), jnp.float32).at[:n].set(x.astype(jnp.float32))

    accum = pl.kernel(
        _accum_body,
        out_type=jax.ShapeDtypeStruct((NC, N_PAD, D), jnp.float32),
        mesh=plsc.VectorSubcoreMesh(core_axis_name="c", subcore_axis_name="s"),
        compiler_params=pltpu.CompilerParams(needs_layout_passes=False),
        scratch_types=[
            pltpu.VMEM_SHARED((N_PAD, D), jnp.float32),
            pltpu.VMEM((NCH, CH), jnp.int32),
            pltpu.VMEM((NCH, CH), jnp.float32),
            pltpu.VMEM((NCH, CH), jnp.int32),
            pltpu.VMEM((CH, D), jnp.float32),
            pltpu.VMEM((CH, D), jnp.float32),
            pltpu.VMEM((CH, D), jnp.float32),
            pltpu.VMEM((CH, D), jnp.float32),
            pltpu.SemaphoreType.DMA,
            pltpu.SemaphoreType.DMA,
            pltpu.SemaphoreType.DMA,
            pltpu.SemaphoreType.DMA,
        ],
    )

    combine = pl.kernel(
        _combine_body,
        out_type=jax.ShapeDtypeStruct((N_PAD * D,), jnp.float32),
        mesh=plsc.VectorSubcoreMesh(core_axis_name="c", subcore_axis_name="s"),
        compiler_params=pltpu.CompilerParams(needs_layout_passes=False),
        scratch_types=[
            pltpu.VMEM((RPW * D,), jnp.float32),
            pltpu.VMEM((RPW * D,), jnp.float32),
            pltpu.VMEM((RPW * D,), jnp.float32),
            pltpu.SemaphoreType.DMA,
            pltpu.SemaphoreType.DMA,
            pltpu.SemaphoreType.DMA,
        ],
    )

    def body(i, xp):
        partials = accum(xp, cs2, vs2, rs2)
        xnf = combine(partials.reshape(NC, N_PAD * D), x0f)
        return xnf.reshape(N_PAD, D)

    xf = lax.fori_loop(0, iter, body, x_pad)
    return xf[:n].astype(x.dtype)
